# Initial kernel scaffold; baseline (speedup 1.0000x reference)
import jax
import jax.numpy as jnp
from jax.experimental import pallas as pl


def kernel(node_features, edge_indices, edge_features, gc1_Wroot, gc1_Wrel, gc1_b, gc2_Wroot, gc2_Wrel, gc2_b, sage_Wl, sage_Wr, sage_b, att_W, att_b, fc_W, fc_b):
    raise NotImplementedError("write your pallas kernel here")


# speedup vs baseline: 3.2398x; 3.2398x over previous
import functools
import jax
import jax.numpy as jnp
from jax import lax
from jax.experimental import pallas as pl
from jax.experimental.pallas import tpu as pltpu
from jax.experimental.pallas import tpu_sc as plsc

N = 10000
E = 320000
D = 128
DE = 16
NC, NS, L = 2, 16, 16
NW = NC * NS
EPW = E // NW
CH = 80
NCHUNK = EPW // CH
ROWS_PT = N // NS
ZR = 125

_MESH = dict(core_axis_name="c", subcore_axis_name="s", num_cores=NC,
             num_subcores=NS)

SB = 80
NSB = N // SB
ZB = 16


def _zero_f32_buf(ref, rows, cols):
    def body(i, _):
        r = i // (cols // L)
        k = i % (cols // L)
        ref[r, pl.ds(k * L, L)] = jnp.zeros((L,), jnp.float32)
        return 0
    lax.fori_loop(0, rows * (cols // L), body, 0)


def _striped(sid, body_fn):
    def body(i, _):
        b = sid + i * NS

        @pl.when(b < NSB)
        def _():
            body_fn(b * SB)
        return 0
    lax.fori_loop(0, (NSB + NS - 1) // NS, body, 0)


def _seg_body(x_hbm, row_hbm, col_hbm, agg_out,
             row_v, col_v, nbuf, zbufD, agg_sh, sem):
    cid = lax.axis_index("c")
    sid = lax.axis_index("s")
    wid = cid * NS + sid

    _zero_f32_buf(zbufD, ZB, D)

    def zero_blk(r0):
        for j in range(SB // ZB):
            pltpu.sync_copy(zbufD, agg_sh.at[pl.ds(r0 + j * ZB, ZB)])
    _striped(sid, zero_blk)

    pltpu.sync_copy(row_hbm.at[wid], row_v)
    pltpu.sync_copy(col_hbm.at[wid], col_v)

    plsc.subcore_barrier()

    def chunk(g, _):
        pltpu.async_copy(x_hbm.at[row_v.at[g]], nbuf, sem).wait()
        pltpu.sync_copy(nbuf, agg_sh.at[col_v.at[g]], add=True)
        return 0
    lax.fori_loop(0, NCHUNK, chunk, 0)

    plsc.subcore_barrier()
    _striped(sid, lambda r0: pltpu.sync_copy(
        agg_sh.at[pl.ds(r0, SB)], agg_out.at[cid, pl.ds(r0, SB)]))


_seg = pl.kernel(
    _seg_body,
    out_type=jax.ShapeDtypeStruct((NC, N, D), jnp.float32),
    mesh=plsc.VectorSubcoreMesh(**_MESH),
    compiler_params=pltpu.CompilerParams(use_tc_tiling_on_sc=False),
    scratch_types=[
        pltpu.VMEM((NCHUNK, CH), jnp.int32),
        pltpu.VMEM((NCHUNK, CH), jnp.int32),
        pltpu.VMEM((CH, D), jnp.float32),
        pltpu.VMEM((ZB, D), jnp.float32),
        pltpu.VMEM_SHARED((N, D), jnp.float32),
        pltpu.SemaphoreType.DMA,
    ],
)


def _k5_body(z_hbm, gp_hbm, cx_hbm, row_hbm, col_hbm, s_out,
             row_v, col_v, cbuf, zbuf, gbuf, zbufD, s_sh, sem):
    cid = lax.axis_index("c")
    sid = lax.axis_index("s")
    wid = cid * NS + sid

    _zero_f32_buf(zbufD, ZB, D)

    def zero_blk(r0):
        for j in range(SB // ZB):
            pltpu.sync_copy(zbufD, s_sh.at[pl.ds(r0 + j * ZB, ZB)])
    _striped(sid, zero_blk)

    pltpu.sync_copy(row_hbm.at[wid], row_v)
    pltpu.sync_copy(col_hbm.at[wid], col_v)

    plsc.subcore_barrier()

    def chunk(g, _):
        pltpu.async_copy(z_hbm.at[row_v.at[g]], zbuf, sem).wait()
        pltpu.sync_copy(
            gp_hbm.at[pl.ds(wid * EPW + g * CH, CH)], gbuf)
        pltpu.sync_copy(
            cx_hbm.at[pl.ds(wid * EPW + g * CH, CH)], cbuf)

        def edge(e, _):
            c16 = cbuf[e]
            for k in range(D // L):
                sl = pl.ds(k * L, L)
                gbuf[e, sl] = gbuf[e, sl] + zbuf[e, sl] * c16
            return 0
        lax.fori_loop(0, CH, edge, 0)

        pltpu.sync_copy(gbuf, s_sh.at[col_v.at[g]], add=True)
        return 0
    lax.fori_loop(0, NCHUNK, chunk, 0)

    plsc.subcore_barrier()
    _striped(sid, lambda r0: pltpu.sync_copy(
        s_sh.at[pl.ds(r0, SB)], s_out.at[cid, pl.ds(r0, SB)]))


_k5 = pl.kernel(
    _k5_body,
    out_type=jax.ShapeDtypeStruct((NC, N, D), jnp.float32),
    mesh=plsc.VectorSubcoreMesh(**_MESH),
    compiler_params=pltpu.CompilerParams(use_tc_tiling_on_sc=False),
    scratch_types=[
        pltpu.VMEM((NCHUNK, CH), jnp.int32),
        pltpu.VMEM((NCHUNK, CH), jnp.int32),
        pltpu.VMEM((CH, L), jnp.float32),
        pltpu.VMEM((CH, D), jnp.float32),
        pltpu.VMEM((CH, D), jnp.float32),
        pltpu.VMEM((ZB, D), jnp.float32),
        pltpu.VMEM_SHARED((N, D), jnp.float32),
        pltpu.SemaphoreType.DMA,
    ],
)


BN = 1000
BE = 2000
NB_E = E // BE


def _k0_body(ef_ref, efz_ref):
    ef = ef_ref[...]
    ones = jnp.ones((BN, DE), jnp.float32)
    zeros = jnp.zeros((BN, D - 2 * DE), jnp.float32)
    efz_ref[...] = jnp.concatenate([ef, ones, zeros], axis=1)


def _run_k0(ef):
    return pl.pallas_call(
        _k0_body,
        grid=(N // BN,),
        in_specs=[pl.BlockSpec((BN, DE), lambda i: (i, 0))],
        out_specs=pl.BlockSpec((BN, D), lambda i: (i, 0)),
        out_shape=jax.ShapeDtypeStruct((N, D), jnp.float32),
    )(ef)


def _k2_body(agg_ref, nf_ref, tc_ref, wrel_ref, wroot_ref, b_ref,
             wl_ref, x1_ref, mt_ref):
    a = agg_ref[0] + agg_ref[1]
    acc = jnp.dot(a, wrel_ref[...], preferred_element_type=jnp.float32)
    acc += jnp.dot(nf_ref[...], wroot_ref[...],
                   preferred_element_type=jnp.float32)
    x1_ref[...] = jnp.maximum(acc + b_ref[...], 0.0)

    tc = tc_ref[0] + tc_ref[1]
    tsum = tc[:, :DE]
    cnt = tc[:, DE]
    mean = tsum / jnp.maximum(cnt, 1.0)[:, None]
    mt_ref[...] = jnp.dot(mean, wl_ref[...], preferred_element_type=jnp.float32)


def _run_k2(agg, nf, tc, wrel, wroot, b, wl):
    return pl.pallas_call(
        _k2_body,
        grid=(N // BN,),
        in_specs=[
            pl.BlockSpec((NC, BN, D), lambda i: (0, i, 0)),
            pl.BlockSpec((BN, D), lambda i: (i, 0)),
            pl.BlockSpec((NC, BN, D), lambda i: (0, i, 0)),
            pl.BlockSpec((D, D), lambda i: (0, 0)),
            pl.BlockSpec((D, D), lambda i: (0, 0)),
            pl.BlockSpec((1, D), lambda i: (0, 0)),
            pl.BlockSpec((DE, D), lambda i: (0, 0)),
        ],
        out_specs=[
            pl.BlockSpec((BN, D), lambda i: (i, 0)),
            pl.BlockSpec((BN, D), lambda i: (i, 0)),
        ],
        out_shape=[
            jax.ShapeDtypeStruct((N, D), jnp.float32),
            jax.ShapeDtypeStruct((N, D), jnp.float32),
        ],
    )(agg, nf, tc, wrel, wroot, b, wl)


def _k2s_body(ef_ref, w_ref, b_ref, s_ref):
    t = jnp.dot(ef_ref[...], w_ref[...],
                preferred_element_type=jnp.float32) + b_ref[...]
    s = jnp.sum(t * t, axis=1)
    s_ref[...] = jnp.where(s >= 0.0, s, 0.01 * s)[None, None, :]


def _run_k2s(ef, att_w, att_b):
    return pl.pallas_call(
        _k2s_body,
        grid=(NB_E,),
        in_specs=[
            pl.BlockSpec((BE, DE), lambda i: (i, 0)),
            pl.BlockSpec((DE, D), lambda i: (0, 0)),
            pl.BlockSpec((1, D), lambda i: (0, 0)),
        ],
        out_specs=pl.BlockSpec((1, 1, BE), lambda i: (i, 0, 0)),
        out_shape=jax.ShapeDtypeStruct((NB_E, 1, BE), jnp.float32),
    )(ef, att_w, att_b)


def _k2c_body(s_ref, c_ref):
    s = s_ref[...]
    m = jnp.max(s)
    ex = jnp.exp(s - m)
    c_ref[...] = ex / jnp.sum(ex)


def _run_k2c(scores):
    return pl.pallas_call(
        _k2c_body,
        out_shape=jax.ShapeDtypeStruct((NB_E, 1, BE), jnp.float32),
    )(scores)


def _k4a_body(agg_ref, x1_ref, wrel_ref, wroot_ref, b_ref, ftop_ref, z_ref):
    a = agg_ref[0] + agg_ref[1]
    acc = jnp.dot(a, wrel_ref[...], preferred_element_type=jnp.float32)
    acc += jnp.dot(x1_ref[...], wroot_ref[...],
                   preferred_element_type=jnp.float32)
    x2 = jnp.maximum(acc + b_ref[...], 0.0)
    z_ref[...] = jnp.dot(x2, ftop_ref[...], preferred_element_type=jnp.float32)


def _run_k4a(agg, x1, wrel, wroot, b, ftop):
    return pl.pallas_call(
        _k4a_body,
        grid=(N // BN,),
        in_specs=[
            pl.BlockSpec((NC, BN, D), lambda i: (0, i, 0)),
            pl.BlockSpec((BN, D), lambda i: (i, 0)),
            pl.BlockSpec((D, D), lambda i: (0, 0)),
            pl.BlockSpec((D, D), lambda i: (0, 0)),
            pl.BlockSpec((1, D), lambda i: (0, 0)),
            pl.BlockSpec((D, D), lambda i: (0, 0)),
        ],
        out_specs=pl.BlockSpec((BN, D), lambda i: (i, 0)),
        out_shape=jax.ShapeDtypeStruct((N, D), jnp.float32),
    )(agg, x1, wrel, wroot, b, ftop)


def _k4b_body(ef_ref, cf_ref, mt_ref, wr_ref, b_ref, fbot_ref, g_ref, cx_ref):
    i = pl.program_id(0)
    pre = jnp.dot(ef_ref[...], wr_ref[...],
                  preferred_element_type=jnp.float32) + b_ref[...]
    mask = jnp.where(i < N // BE, 1.0, 0.0)
    e = jnp.maximum(pre + mask * mt_ref[...], 0.0)
    cf = cf_ref[0, 0, :][:, None]
    w = cf * e
    g_ref[...] = jnp.dot(w, fbot_ref[...], preferred_element_type=jnp.float32)
    cx_ref[...] = jnp.broadcast_to(cf, (BE, L))


def _run_k4b(ef, coeffs, mt, wr, b, fbot):
    return pl.pallas_call(
        _k4b_body,
        grid=(NB_E,),
        in_specs=[
            pl.BlockSpec((BE, DE), lambda i: (i, 0)),
            pl.BlockSpec((1, 1, BE), lambda i: (i, 0, 0)),
            pl.BlockSpec((BE, D), lambda i: (jnp.minimum(i, N // BE - 1), 0)),
            pl.BlockSpec((DE, D), lambda i: (0, 0)),
            pl.BlockSpec((1, D), lambda i: (0, 0)),
            pl.BlockSpec((D, D), lambda i: (0, 0)),
        ],
        out_specs=[
            pl.BlockSpec((BE, D), lambda i: (i, 0)),
            pl.BlockSpec((BE, L), lambda i: (i, 0)),
        ],
        out_shape=[
            jax.ShapeDtypeStruct((E, D), jnp.float32),
            jax.ShapeDtypeStruct((E, L), jnp.float32),
        ],
    )(ef, coeffs, mt, wr, b, fbot)


def _k6_body(z_ref, s_ref, b_ref, o_ref):
    o_ref[...] = z_ref[...] + s_ref[0] + s_ref[1] + b_ref[...]


def _run_k6(z, s, b):
    return pl.pallas_call(
        _k6_body,
        grid=(N // BN,),
        in_specs=[
            pl.BlockSpec((BN, D), lambda i: (i, 0)),
            pl.BlockSpec((NC, BN, D), lambda i: (0, i, 0)),
            pl.BlockSpec((1, D), lambda i: (0, 0)),
        ],
        out_specs=pl.BlockSpec((BN, D), lambda i: (i, 0)),
        out_shape=jax.ShapeDtypeStruct((N, D), jnp.float32),
    )(z, s, b)


@jax.jit
def kernel(node_features, edge_indices, edge_features,
           gc1_Wroot, gc1_Wrel, gc1_b,
           gc2_Wroot, gc2_Wrel, gc2_b,
           sage_Wl, sage_Wr, sage_b,
           att_W, att_b, fc_W, fc_b):
    row2 = edge_indices[0].reshape(NW, NCHUNK, CH)
    col2 = edge_indices[1].reshape(NW, NCHUNK, CH)
    b1 = gc1_b.reshape(1, D)
    b2 = gc2_b.reshape(1, D)
    bs = sage_b.reshape(1, D)
    ba = att_b.reshape(1, D)
    bf = fc_b.reshape(1, D)
    fc_top = fc_W[:D]
    fc_bot = fc_W[D:]

    efz = _run_k0(edge_features)
    agg1 = _seg(node_features, row2, col2)
    tc_acc = _seg(efz, row2, col2)
    x1, mt = _run_k2(agg1, node_features, tc_acc,
                     gc1_Wrel, gc1_Wroot, b1, sage_Wl)
    scores = _run_k2s(edge_features, att_W, ba)
    coeffs = _run_k2c(scores)
    agg2 = _seg(x1, row2, col2)
    z = _run_k4a(agg2, x1, gc2_Wrel, gc2_Wroot, b2, fc_top)
    gp, cx = _run_k4b(edge_features, coeffs, mt, sage_Wr, bs, fc_bot)
    s_acc = _k5(z, gp, cx, row2, col2)
    return _run_k6(z, s_acc, bf)


# speedup vs baseline: 3.4102x; 1.0526x over previous
import functools
import jax
import jax.numpy as jnp
from jax import lax
from jax.experimental import pallas as pl
from jax.experimental.pallas import tpu as pltpu
from jax.experimental.pallas import tpu_sc as plsc

N = 10000
E = 320000
D = 128
DE = 16
NC, NS, L = 2, 16, 16
NW = NC * NS
EPW = E // NW
CH = 80
NCHUNK = EPW // CH
ROWS_PT = N // NS
ZR = 125

_MESH = dict(core_axis_name="c", subcore_axis_name="s", num_cores=NC,
             num_subcores=NS)

SB = 80
NSB = N // SB
ZB = 16


def _zero_f32_buf(ref, rows, cols):
    def body(i, _):
        r = i // (cols // L)
        k = i % (cols // L)
        ref[r, pl.ds(k * L, L)] = jnp.zeros((L,), jnp.float32)
        return 0
    lax.fori_loop(0, rows * (cols // L), body, 0)


def _striped(sid, body_fn):
    def body(i, _):
        b = sid + i * NS

        @pl.when(b < NSB)
        def _():
            body_fn(b * SB)
        return 0
    lax.fori_loop(0, (NSB + NS - 1) // NS, body, 0)


def _seg_body(x_hbm, row_hbm, col_hbm, agg_out,
              row_v, col_v, nb0, nb1, zbufD, agg_sh, sem0, sem1):
    cid = lax.axis_index("c")
    sid = lax.axis_index("s")
    wid = cid * NS + sid

    _zero_f32_buf(zbufD, ZB, D)

    def zero_blk(r0):
        for j in range(SB // ZB):
            pltpu.sync_copy(zbufD, agg_sh.at[pl.ds(r0 + j * ZB, ZB)])
    _striped(sid, zero_blk)

    pltpu.sync_copy(row_hbm.at[wid], row_v)
    pltpu.sync_copy(col_hbm.at[wid], col_v)

    plsc.subcore_barrier()

    pltpu.async_copy(x_hbm.at[row_v.at[0]], nb0, sem0)

    def pair(i, _):
        g0 = 2 * i
        pltpu.make_async_copy(x_hbm.at[row_v.at[g0]], nb0, sem0).wait()
        pltpu.async_copy(x_hbm.at[row_v.at[g0 + 1]], nb1, sem1)
        pltpu.sync_copy(nb0, agg_sh.at[col_v.at[g0]], add=True)
        pltpu.make_async_copy(x_hbm.at[row_v.at[g0 + 1]], nb1, sem1).wait()
        pltpu.async_copy(x_hbm.at[row_v.at[g0 + 2]], nb0, sem0)
        pltpu.sync_copy(nb1, agg_sh.at[col_v.at[g0 + 1]], add=True)
        return 0
    lax.fori_loop(0, NCHUNK // 2, pair, 0)

    pltpu.make_async_copy(x_hbm.at[row_v.at[NCHUNK - 1]], nb0, sem0).wait()
    pltpu.sync_copy(nb0, agg_sh.at[col_v.at[NCHUNK - 1]], add=True)

    plsc.subcore_barrier()
    _striped(sid, lambda r0: pltpu.sync_copy(
        agg_sh.at[pl.ds(r0, SB)], agg_out.at[cid, pl.ds(r0, SB)]))


_seg = pl.kernel(
    _seg_body,
    out_type=jax.ShapeDtypeStruct((NC, N, D), jnp.float32),
    mesh=plsc.VectorSubcoreMesh(**_MESH),
    compiler_params=pltpu.CompilerParams(use_tc_tiling_on_sc=False),
    scratch_types=[
        pltpu.VMEM((NCHUNK, CH), jnp.int32),
        pltpu.VMEM((NCHUNK, CH), jnp.int32),
        pltpu.VMEM((CH, D), jnp.float32),
        pltpu.VMEM((CH, D), jnp.float32),
        pltpu.VMEM((ZB, D), jnp.float32),
        pltpu.VMEM_SHARED((N, D), jnp.float32),
        pltpu.SemaphoreType.DMA,
        pltpu.SemaphoreType.DMA,
    ],
)


def _k5_body(z_hbm, gp_hbm, cx_hbm, row_hbm, col_hbm, s_out,
             rb0, rb1, cb0, cb1, cbuf, zb0, zb1, gbuf, s_sh, sem0, sem1):
    cid = lax.axis_index("c")
    sid = lax.axis_index("s")
    wid = cid * NS + sid

    _zero_f32_buf(gbuf, SB, D)

    def zero_blk(r0):
        pltpu.sync_copy(gbuf, s_sh.at[pl.ds(r0, SB)])
    _striped(sid, zero_blk)

    plsc.subcore_barrier()

    def compute(zbuf):
        def quad(q, _):
            for u in range(4):
                e = q * 4 + u
                c16 = cbuf[e]
                for k in range(D // L):
                    sl = pl.ds(k * L, L)
                    gbuf[e, sl] = gbuf[e, sl] + zbuf[e, sl] * c16
            return 0
        lax.fori_loop(0, CH // 4, quad, 0)

    def load_gp_cx(g):
        pltpu.sync_copy(gp_hbm.at[pl.ds(wid * EPW + g * CH, CH)], gbuf)
        pltpu.sync_copy(cx_hbm.at[pl.ds(wid * EPW + g * CH, CH)], cbuf)

    pltpu.sync_copy(row_hbm.at[wid, 0], rb0)
    pltpu.sync_copy(col_hbm.at[wid, 0], cb0)
    pltpu.async_copy(z_hbm.at[rb0], zb0, sem0)
    pltpu.sync_copy(row_hbm.at[wid, 1], rb1)
    pltpu.sync_copy(col_hbm.at[wid, 1], cb1)

    def pair(i, _):
        g0 = 2 * i
        pltpu.async_copy(z_hbm.at[rb1], zb1, sem1)
        load_gp_cx(g0)
        pltpu.make_async_copy(z_hbm.at[rb0], zb0, sem0).wait()
        compute(zb0)
        pltpu.sync_copy(gbuf, s_sh.at[cb0], add=True)
        pltpu.sync_copy(row_hbm.at[wid, g0 + 2], rb0)
        pltpu.sync_copy(col_hbm.at[wid, g0 + 2], cb0)
        pltpu.async_copy(z_hbm.at[rb0], zb0, sem0)

        load_gp_cx(g0 + 1)
        pltpu.make_async_copy(z_hbm.at[rb1], zb1, sem1).wait()
        compute(zb1)
        pltpu.sync_copy(gbuf, s_sh.at[cb1], add=True)

        @pl.when(g0 + 3 < NCHUNK)
        def _():
            pltpu.sync_copy(row_hbm.at[wid, g0 + 3], rb1)
            pltpu.sync_copy(col_hbm.at[wid, g0 + 3], cb1)
        return 0
    lax.fori_loop(0, NCHUNK // 2, pair, 0)

    load_gp_cx(NCHUNK - 1)
    pltpu.make_async_copy(z_hbm.at[rb0], zb0, sem0).wait()
    compute(zb0)
    pltpu.sync_copy(gbuf, s_sh.at[cb0], add=True)

    plsc.subcore_barrier()
    _striped(sid, lambda r0: pltpu.sync_copy(
        s_sh.at[pl.ds(r0, SB)], s_out.at[cid, pl.ds(r0, SB)]))


_k5 = pl.kernel(
    _k5_body,
    out_type=jax.ShapeDtypeStruct((NC, N, D), jnp.float32),
    mesh=plsc.VectorSubcoreMesh(**_MESH),
    compiler_params=pltpu.CompilerParams(use_tc_tiling_on_sc=False),
    scratch_types=[
        pltpu.VMEM((CH,), jnp.int32),
        pltpu.VMEM((CH,), jnp.int32),
        pltpu.VMEM((CH,), jnp.int32),
        pltpu.VMEM((CH,), jnp.int32),
        pltpu.VMEM((CH, L), jnp.float32),
        pltpu.VMEM((CH, D), jnp.float32),
        pltpu.VMEM((CH, D), jnp.float32),
        pltpu.VMEM((CH, D), jnp.float32),
        pltpu.VMEM_SHARED((N, D), jnp.float32),
        pltpu.SemaphoreType.DMA,
        pltpu.SemaphoreType.DMA,
    ],
)


BN = 1000
BE = 2000
NB_E = E // BE


def _k0_body(ef_ref, efz_ref):
    ef = ef_ref[...]
    ones = jnp.ones((BN, DE), jnp.float32)
    zeros = jnp.zeros((BN, D - 2 * DE), jnp.float32)
    efz_ref[...] = jnp.concatenate([ef, ones, zeros], axis=1)


def _run_k0(ef):
    return pl.pallas_call(
        _k0_body,
        grid=(N // BN,),
        in_specs=[pl.BlockSpec((BN, DE), lambda i: (i, 0))],
        out_specs=pl.BlockSpec((BN, D), lambda i: (i, 0)),
        out_shape=jax.ShapeDtypeStruct((N, D), jnp.float32),
    )(ef)


def _k2_body(agg_ref, nf_ref, tc_ref, wrel_ref, wroot_ref, b_ref,
             wl_ref, x1_ref, mt_ref):
    a = agg_ref[0] + agg_ref[1]
    acc = jnp.dot(a, wrel_ref[...], preferred_element_type=jnp.float32)
    acc += jnp.dot(nf_ref[...], wroot_ref[...],
                   preferred_element_type=jnp.float32)
    x1_ref[...] = jnp.maximum(acc + b_ref[...], 0.0)

    tc = tc_ref[0] + tc_ref[1]
    tsum = tc[:, :DE]
    cnt = tc[:, DE]
    mean = tsum / jnp.maximum(cnt, 1.0)[:, None]
    mt_ref[...] = jnp.dot(mean, wl_ref[...], preferred_element_type=jnp.float32)


def _run_k2(agg, nf, tc, wrel, wroot, b, wl):
    return pl.pallas_call(
        _k2_body,
        grid=(N // BN,),
        in_specs=[
            pl.BlockSpec((NC, BN, D), lambda i: (0, i, 0)),
            pl.BlockSpec((BN, D), lambda i: (i, 0)),
            pl.BlockSpec((NC, BN, D), lambda i: (0, i, 0)),
            pl.BlockSpec((D, D), lambda i: (0, 0)),
            pl.BlockSpec((D, D), lambda i: (0, 0)),
            pl.BlockSpec((1, D), lambda i: (0, 0)),
            pl.BlockSpec((DE, D), lambda i: (0, 0)),
        ],
        out_specs=[
            pl.BlockSpec((BN, D), lambda i: (i, 0)),
            pl.BlockSpec((BN, D), lambda i: (i, 0)),
        ],
        out_shape=[
            jax.ShapeDtypeStruct((N, D), jnp.float32),
            jax.ShapeDtypeStruct((N, D), jnp.float32),
        ],
    )(agg, nf, tc, wrel, wroot, b, wl)


def _k2s_body(ef_ref, w_ref, b_ref, s_ref):
    t = jnp.dot(ef_ref[...], w_ref[...],
                preferred_element_type=jnp.float32) + b_ref[...]
    s = jnp.sum(t * t, axis=1)
    s_ref[...] = jnp.where(s >= 0.0, s, 0.01 * s)[None, None, :]


def _run_k2s(ef, att_w, att_b):
    return pl.pallas_call(
        _k2s_body,
        grid=(NB_E,),
        in_specs=[
            pl.BlockSpec((BE, DE), lambda i: (i, 0)),
            pl.BlockSpec((DE, D), lambda i: (0, 0)),
            pl.BlockSpec((1, D), lambda i: (0, 0)),
        ],
        out_specs=pl.BlockSpec((1, 1, BE), lambda i: (i, 0, 0)),
        out_shape=jax.ShapeDtypeStruct((NB_E, 1, BE), jnp.float32),
    )(ef, att_w, att_b)


def _k2c_body(s_ref, c_ref):
    s = s_ref[...]
    m = jnp.max(s)
    ex = jnp.exp(s - m)
    c_ref[...] = ex / jnp.sum(ex)


def _run_k2c(scores):
    return pl.pallas_call(
        _k2c_body,
        out_shape=jax.ShapeDtypeStruct((NB_E, 1, BE), jnp.float32),
    )(scores)


def _k4a_body(agg_ref, x1_ref, wrel_ref, wroot_ref, b_ref, ftop_ref, z_ref):
    a = agg_ref[0] + agg_ref[1]
    acc = jnp.dot(a, wrel_ref[...], preferred_element_type=jnp.float32)
    acc += jnp.dot(x1_ref[...], wroot_ref[...],
                   preferred_element_type=jnp.float32)
    x2 = jnp.maximum(acc + b_ref[...], 0.0)
    z_ref[...] = jnp.dot(x2, ftop_ref[...], preferred_element_type=jnp.float32)


def _run_k4a(agg, x1, wrel, wroot, b, ftop):
    return pl.pallas_call(
        _k4a_body,
        grid=(N // BN,),
        in_specs=[
            pl.BlockSpec((NC, BN, D), lambda i: (0, i, 0)),
            pl.BlockSpec((BN, D), lambda i: (i, 0)),
            pl.BlockSpec((D, D), lambda i: (0, 0)),
            pl.BlockSpec((D, D), lambda i: (0, 0)),
            pl.BlockSpec((1, D), lambda i: (0, 0)),
            pl.BlockSpec((D, D), lambda i: (0, 0)),
        ],
        out_specs=pl.BlockSpec((BN, D), lambda i: (i, 0)),
        out_shape=jax.ShapeDtypeStruct((N, D), jnp.float32),
    )(agg, x1, wrel, wroot, b, ftop)


def _k4b_body(ef_ref, cf_ref, mt_ref, wr_ref, b_ref, fbot_ref, g_ref, cx_ref):
    i = pl.program_id(0)
    pre = jnp.dot(ef_ref[...], wr_ref[...],
                  preferred_element_type=jnp.float32) + b_ref[...]
    mask = jnp.where(i < N // BE, 1.0, 0.0)
    e = jnp.maximum(pre + mask * mt_ref[...], 0.0)
    cf = cf_ref[0, 0, :][:, None]
    w = cf * e
    g_ref[...] = jnp.dot(w, fbot_ref[...], preferred_element_type=jnp.float32)
    cx_ref[...] = jnp.broadcast_to(cf, (BE, L))


def _run_k4b(ef, coeffs, mt, wr, b, fbot):
    return pl.pallas_call(
        _k4b_body,
        grid=(NB_E,),
        in_specs=[
            pl.BlockSpec((BE, DE), lambda i: (i, 0)),
            pl.BlockSpec((1, 1, BE), lambda i: (i, 0, 0)),
            pl.BlockSpec((BE, D), lambda i: (jnp.minimum(i, N // BE - 1), 0)),
            pl.BlockSpec((DE, D), lambda i: (0, 0)),
            pl.BlockSpec((1, D), lambda i: (0, 0)),
            pl.BlockSpec((D, D), lambda i: (0, 0)),
        ],
        out_specs=[
            pl.BlockSpec((BE, D), lambda i: (i, 0)),
            pl.BlockSpec((BE, L), lambda i: (i, 0)),
        ],
        out_shape=[
            jax.ShapeDtypeStruct((E, D), jnp.float32),
            jax.ShapeDtypeStruct((E, L), jnp.float32),
        ],
    )(ef, coeffs, mt, wr, b, fbot)


def _k6_body(z_ref, s_ref, b_ref, o_ref):
    o_ref[...] = z_ref[...] + s_ref[0] + s_ref[1] + b_ref[...]


def _run_k6(z, s, b):
    return pl.pallas_call(
        _k6_body,
        grid=(N // BN,),
        in_specs=[
            pl.BlockSpec((BN, D), lambda i: (i, 0)),
            pl.BlockSpec((NC, BN, D), lambda i: (0, i, 0)),
            pl.BlockSpec((1, D), lambda i: (0, 0)),
        ],
        out_specs=pl.BlockSpec((BN, D), lambda i: (i, 0)),
        out_shape=jax.ShapeDtypeStruct((N, D), jnp.float32),
    )(z, s, b)


@jax.jit
def kernel(node_features, edge_indices, edge_features,
           gc1_Wroot, gc1_Wrel, gc1_b,
           gc2_Wroot, gc2_Wrel, gc2_b,
           sage_Wl, sage_Wr, sage_b,
           att_W, att_b, fc_W, fc_b):
    row2 = edge_indices[0].reshape(NW, NCHUNK, CH)
    col2 = edge_indices[1].reshape(NW, NCHUNK, CH)
    b1 = gc1_b.reshape(1, D)
    b2 = gc2_b.reshape(1, D)
    bs = sage_b.reshape(1, D)
    ba = att_b.reshape(1, D)
    bf = fc_b.reshape(1, D)
    fc_top = fc_W[:D]
    fc_bot = fc_W[D:]

    efz = _run_k0(edge_features)
    agg1 = _seg(node_features, row2, col2)
    tc_acc = _seg(efz, row2, col2)
    x1, mt = _run_k2(agg1, node_features, tc_acc,
                     gc1_Wrel, gc1_Wroot, b1, sage_Wl)
    scores = _run_k2s(edge_features, att_W, ba)
    coeffs = _run_k2c(scores)
    agg2 = _seg(x1, row2, col2)
    z = _run_k4a(agg2, x1, gc2_Wrel, gc2_Wroot, b2, fc_top)
    gp, cx = _run_k4b(edge_features, coeffs, mt, sage_Wr, bs, fc_bot)
    s_acc = _k5(z, gp, cx, row2, col2)
    return _run_k6(z, s_acc, bf)
